# baseline (speedup 1.0000x reference)
import functools

import jax
import jax.numpy as jnp
from jax import lax
from jax.experimental import pallas as pl
from jax.experimental.pallas import tpu as pltpu
from jax.experimental.pallas import tpu_sc as plsc

B = 16384
D = 64
NC = 2
NS = 16
NW = NC * NS
B_PER_W = B // NW
CHUNK = 128
NCHUNK = B_PER_W // CHUNK

_mesh = plsc.VectorSubcoreMesh(core_axis_name="c", subcore_axis_name="s")


@functools.partial(
    pl.kernel,
    mesh=_mesh,
    out_type=jax.ShapeDtypeStruct((B, D), jnp.float32),
    scratch_types=[
        pltpu.VMEM((NCHUNK, CHUNK), jnp.int32),
        pltpu.VMEM((B_PER_W, D), jnp.float32),
        pltpu.SemaphoreType.DMA,
    ],
    compiler_params=pltpu.CompilerParams(use_tc_tiling_on_sc=False),
)
def _gather(t_hbm, emb_hbm, out_hbm, idx_v, rows_v, sem):
    wid = lax.axis_index("s") * NC + lax.axis_index("c")
    base = wid * B_PER_W
    for j in range(NCHUNK):
        pltpu.sync_copy(
            t_hbm.at[pl.ds(base + j * CHUNK, CHUNK)],
            idx_v.at[j],
        )
    for j in range(NCHUNK):
        pltpu.async_copy(
            emb_hbm.at[idx_v.at[j]],
            rows_v.at[pl.ds(j * CHUNK, CHUNK)],
            sem,
        )
    for j in range(NCHUNK):
        pltpu.make_async_copy(
            emb_hbm.at[idx_v.at[j]],
            rows_v.at[pl.ds(j * CHUNK, CHUNK)],
            sem,
        ).wait()
    pltpu.sync_copy(rows_v, out_hbm.at[pl.ds(base, B_PER_W)])


def kernel(t, emb):
    return _gather(t, emb)


# speedup vs baseline: 1.1480x; 1.1480x over previous
import functools

import jax
import jax.numpy as jnp
from jax import lax
from jax.experimental import pallas as pl
from jax.experimental.pallas import tpu as pltpu
from jax.experimental.pallas import tpu_sc as plsc

B = 16384
D = 64
DP = 128
VP = 100008
NC = 2
NS = 16
NW = NC * NS
B_PER_W = B // NW
CHUNK = 128
NCHUNK = B_PER_W // CHUNK

_mesh = plsc.VectorSubcoreMesh(core_axis_name="c", subcore_axis_name="s")


@functools.partial(
    pl.kernel,
    mesh=_mesh,
    out_type=jax.ShapeDtypeStruct((B, DP), jnp.float32),
    scratch_types=[
        pltpu.VMEM((NCHUNK, CHUNK), jnp.int32),
        pltpu.VMEM((B_PER_W, DP), jnp.float32),
        pltpu.SemaphoreType.DMA,
    ],
)
def _gather(t_hbm, emb_hbm, out_hbm, idx_v, rows_v, sem):
    wid = lax.axis_index("s") * NC + lax.axis_index("c")
    base = wid * B_PER_W
    for j in range(NCHUNK):
        pltpu.sync_copy(
            t_hbm.at[pl.ds(base + j * CHUNK, CHUNK)],
            idx_v.at[j],
        )
    for j in range(NCHUNK):
        pltpu.async_copy(
            emb_hbm.at[idx_v.at[j]],
            rows_v.at[pl.ds(j * CHUNK, CHUNK)],
            sem,
        )
    for j in range(NCHUNK):
        pltpu.make_async_copy(
            emb_hbm.at[idx_v.at[j]],
            rows_v.at[pl.ds(j * CHUNK, CHUNK)],
            sem,
        ).wait()
    pltpu.sync_copy(rows_v, out_hbm.at[pl.ds(base, B_PER_W)])


def kernel(t, emb):
    emb_p = jnp.pad(emb, ((0, VP - emb.shape[0]), (0, DP - D)))
    return _gather(t, emb_p)[:, :D]


# speedup vs baseline: 1.3727x; 1.1957x over previous
import functools

import jax
import jax.numpy as jnp
from jax import lax
from jax.experimental import pallas as pl
from jax.experimental.pallas import tpu as pltpu
from jax.experimental.pallas import tpu_sc as plsc

B = 16384
D = 64
DP = 128
V = 100001
VP = 100008
NC = 2
NS = 16
NW = NC * NS
B_PER_W = B // NW
CHUNK = 128
NCHUNK = B_PER_W // CHUNK

TBLK = 4096
TGRID = -(-V // TBLK)


def _transpose_body(embt_ref, out_ref):
    x = embt_ref[...]
    y = x.T
    out_ref[...] = jnp.concatenate([y, y], axis=1)


def _transpose_pad(embt):
    return pl.pallas_call(
        _transpose_body,
        grid=(TGRID,),
        in_specs=[pl.BlockSpec((D, TBLK), lambda j: (0, j))],
        out_specs=pl.BlockSpec((TBLK, DP), lambda j: (j, 0)),
        out_shape=jax.ShapeDtypeStruct((VP, DP), jnp.float32),
    )(embt)


_mesh = plsc.VectorSubcoreMesh(core_axis_name="c", subcore_axis_name="s")


@functools.partial(
    pl.kernel,
    mesh=_mesh,
    out_type=jax.ShapeDtypeStruct((B, DP), jnp.float32),
    scratch_types=[
        pltpu.VMEM((NCHUNK, CHUNK), jnp.int32),
        pltpu.VMEM((B_PER_W, DP), jnp.float32),
        pltpu.SemaphoreType.DMA,
    ],
)
def _gather(t_hbm, emb_hbm, out_hbm, idx_v, rows_v, sem):
    wid = lax.axis_index("s") * NC + lax.axis_index("c")
    base = wid * B_PER_W
    for j in range(NCHUNK):
        pltpu.sync_copy(
            t_hbm.at[pl.ds(base + j * CHUNK, CHUNK)],
            idx_v.at[j],
        )
    for j in range(NCHUNK):
        pltpu.async_copy(
            emb_hbm.at[idx_v.at[j]],
            rows_v.at[pl.ds(j * CHUNK, CHUNK)],
            sem,
        )
    for j in range(NCHUNK):
        pltpu.make_async_copy(
            emb_hbm.at[idx_v.at[j]],
            rows_v.at[pl.ds(j * CHUNK, CHUNK)],
            sem,
        ).wait()
    pltpu.sync_copy(rows_v, out_hbm.at[pl.ds(base, B_PER_W)])


def kernel(t, emb):
    emb_p = _transpose_pad(emb.T)
    return _gather(t, emb_p)[:, :D]
